# baseline (speedup 1.0000x reference)
import functools

import jax
import jax.numpy as jnp
from jax.experimental import pallas as pl
from jax.experimental.pallas import tpu as pltpu

NUM_EXPERTS = 8
TOP_K = 2
N_NORMAL = 5
LANES = 128
TOK_BLK = 256
FF_CHUNK = 256


def _router_body(x_ref, wgcat_ref, cvec_ref, logits_ref, gdense_ref, gmeta_ref,
                 base_ref):
    xb = x_ref[...]
    lg = jax.lax.dot_general(
        xb, wgcat_ref[...], (((1,), (1,)), ((), ())),
        preferred_element_type=jnp.float32,
        precision=jax.lax.Precision.DEFAULT)
    logits_ref[...] = lg

    lane = jax.lax.broadcasted_iota(jnp.int32, lg.shape, 1)
    valid = lane < NUM_EXPERTS
    neg = jnp.float32(-1e30)
    l8 = jnp.where(valid, lg, neg)
    m = jnp.max(l8, axis=1, keepdims=True)
    ex = jnp.where(valid, jnp.exp(l8 - m), 0.0)
    s = jnp.sum(ex, axis=1, keepdims=True)
    p = ex / s

    v1 = jnp.max(p, axis=1, keepdims=True)
    big = jnp.int32(LANES)
    i1 = jnp.min(jnp.where(valid & (p == v1), lane, big), axis=1, keepdims=True)
    cand = jnp.where(valid & (lane != i1), p, -1.0)
    v2 = jnp.max(cand, axis=1, keepdims=True)
    i2 = jnp.min(jnp.where(cand == v2, lane, big), axis=1, keepdims=True)

    zero_id = jnp.int32(NUM_EXPERTS - 1)
    gA = jnp.where(i1 == zero_id, 0.0, v1)
    gB = jnp.where(i2 == zero_id, 0.0, v2)
    ssum = gA + gB
    gA = gA / ssum
    gB = gB / ssum

    nrm = jnp.int32(N_NORMAL)
    g0 = jnp.where(i1 < nrm, gA, 0.0)
    g1 = jnp.where(i2 < nrm, gB, 0.0)
    d5 = jnp.where(i1 == nrm, gA, 0.0) + jnp.where(i2 == nrm, gB, 0.0)
    d6 = (jnp.where(i1 == nrm + 1, gA, 0.0)
          + jnp.where(i2 == nrm + 1, gB, 0.0))

    gdense_ref[...] = (jnp.where(lane == i1, gA, 0.0)
                       + jnp.where(lane == i2, gB, 0.0))
    gmeta_ref[...] = (jnp.where(lane == 0, g0, 0.0)
                      + jnp.where(lane == 1, g1, 0.0)
                      + jnp.where(lane == 2, i1.astype(jnp.float32), 0.0)
                      + jnp.where(lane == 3, i2.astype(jnp.float32), 0.0))

    c0 = lg[:, NUM_EXPERTS:NUM_EXPERTS + 1] * 2.0
    c1 = lg[:, NUM_EXPERTS + 1:NUM_EXPERTS + 2] * 2.0
    cm = jnp.maximum(c0, c1)
    e0 = jnp.exp(c0 - cm)
    e1 = jnp.exp(c1 - cm)
    cw0 = e0 / (e0 + e1)
    cw1 = e1 / (e0 + e1)
    x2 = xb * 2.0
    ceo = cw0 * x2 + cw1 * cvec_ref[0:1, :]
    base_ref[...] = d5 * ceo + d6 * x2


def _router_call(xt, wgcat, cvec):
    T, D = xt.shape
    grid = (T // TOK_BLK,)
    return pl.pallas_call(
        _router_body,
        grid=grid,
        in_specs=[
            pl.BlockSpec((TOK_BLK, D), lambda t: (t, 0)),
            pl.BlockSpec((LANES, D), lambda t: (0, 0)),
            pl.BlockSpec((8, D), lambda t: (0, 0)),
        ],
        out_specs=[
            pl.BlockSpec((TOK_BLK, LANES), lambda t: (t, 0)),
            pl.BlockSpec((TOK_BLK, LANES), lambda t: (t, 0)),
            pl.BlockSpec((TOK_BLK, LANES), lambda t: (t, 0)),
            pl.BlockSpec((TOK_BLK, D), lambda t: (t, 0)),
        ],
        out_shape=[
            jax.ShapeDtypeStruct((T, LANES), jnp.float32),
            jax.ShapeDtypeStruct((T, LANES), jnp.float32),
            jax.ShapeDtypeStruct((T, LANES), jnp.float32),
            jax.ShapeDtypeStruct((T, D), jnp.float32),
        ],
    )(xt, wgcat, cvec)


def _ffn_body(gd_ref, x_ref, w1_ref, w2_ref, out_ref):
    e = pl.program_id(0)
    f = pl.program_id(1)
    t = pl.program_id(2)
    sl = pl.ds(t * TOK_BLK, TOK_BLK)
    xb = x_ref[sl, :] * jnp.bfloat16(2.0)
    w1c = w1_ref[0]
    h = jax.lax.dot_general(xb, w1c, (((1,), (1,)), ((), ())),
                            preferred_element_type=jnp.float32)
    h = jax.nn.gelu(h)
    hb = h.astype(jnp.bfloat16)
    w2c = w2_ref[0]
    o = jax.lax.dot_general(hb, w2c, (((1,), (1,)), ((), ())),
                            preferred_element_type=jnp.float32)
    gd = gd_ref[sl, :]
    lane = jax.lax.broadcasted_iota(jnp.int32, gd.shape, 1)
    g = jnp.sum(jnp.where(lane == e, gd, 0.0), axis=1, keepdims=True)
    contrib = g * o

    @pl.when((e == 0) & (f == 0))
    def _init():
        out_ref[sl, :] = contrib

    @pl.when((e > 0) | (f > 0))
    def _acc():
        out_ref[sl, :] += contrib


def _ffn_call(gdense, x_bf, w1_bf, w2_bf):
    T, D = x_bf.shape
    E, F, _ = w1_bf.shape
    grid = (E, F // FF_CHUNK, T // TOK_BLK)
    return pl.pallas_call(
        _ffn_body,
        grid=grid,
        in_specs=[
            pl.BlockSpec((T, LANES), lambda e, f, t: (0, 0)),
            pl.BlockSpec((T, D), lambda e, f, t: (0, 0)),
            pl.BlockSpec((1, FF_CHUNK, D), lambda e, f, t: (e, f, 0)),
            pl.BlockSpec((1, D, FF_CHUNK), lambda e, f, t: (e, 0, f)),
        ],
        out_specs=pl.BlockSpec((T, D), lambda e, f, t: (0, 0)),
        out_shape=jax.ShapeDtypeStruct((T, D), jnp.float32),
    )(gdense, x_bf, w1_bf, w2_bf)


def kernel(x, wg, w1, w2, const_vec, const_wg):
    B, S, D = x.shape
    T = B * S
    xt = x.reshape(T, D)
    wgcat = jnp.zeros((LANES, D), jnp.float32)
    wgcat = wgcat.at[0:NUM_EXPERTS].set(wg)
    wgcat = wgcat.at[NUM_EXPERTS:NUM_EXPERTS + 2].set(const_wg)
    cvec = jnp.broadcast_to(const_vec[None, :], (8, D))

    logits_pad, gdense, gmeta, base = _router_call(xt, wgcat, cvec)
    logits = logits_pad[:, :NUM_EXPERTS]

    x_bf = xt.astype(jnp.bfloat16)
    w1_bf = w1.astype(jnp.bfloat16)
    w2_bf = w2.astype(jnp.bfloat16)
    out = base + _ffn_call(gdense, x_bf, w1_bf, w2_bf)
    return out.reshape(x.shape), logits


# speedup vs baseline: 2.2325x; 2.2325x over previous
import functools

import jax
import jax.numpy as jnp
from jax import lax
from jax.experimental import pallas as pl
from jax.experimental.pallas import tpu as pltpu
from jax.experimental.pallas import tpu_sc as plsc

NUM_EXPERTS = 8
N_NORMAL = 5
LANES = 128
TOK_BLK = 256
BM = 512
FF_CHUNK = 512
NC, NS = 2, 16
NW = NC * NS


def _router_body(x_ref, wgcat_ref, cvec_ref, logits_ref, gmeta_ref, base_ref):
    xb = x_ref[...]
    lg = jax.lax.dot_general(
        xb, wgcat_ref[...], (((1,), (1,)), ((), ())),
        preferred_element_type=jnp.float32,
        precision=jax.lax.Precision.DEFAULT)
    logits_ref[...] = lg

    lane = jax.lax.broadcasted_iota(jnp.int32, lg.shape, 1)
    valid = lane < NUM_EXPERTS
    l8 = jnp.where(valid, lg, jnp.float32(-1e30))
    m = jnp.max(l8, axis=1, keepdims=True)
    ex = jnp.where(valid, jnp.exp(l8 - m), 0.0)
    p = ex / jnp.sum(ex, axis=1, keepdims=True)

    big = jnp.int32(LANES)
    v1 = jnp.max(p, axis=1, keepdims=True)
    i1 = jnp.min(jnp.where(valid & (p == v1), lane, big), axis=1, keepdims=True)
    cand = jnp.where(valid & (lane != i1), p, -1.0)
    v2 = jnp.max(cand, axis=1, keepdims=True)
    i2 = jnp.min(jnp.where(cand == v2, lane, big), axis=1, keepdims=True)

    zero_id = jnp.int32(NUM_EXPERTS - 1)
    gA = jnp.where(i1 == zero_id, 0.0, v1)
    gB = jnp.where(i2 == zero_id, 0.0, v2)
    ssum = gA + gB
    gA = gA / ssum
    gB = gB / ssum

    nrm = jnp.int32(N_NORMAL)
    g0 = jnp.where(i1 < nrm, gA, 0.0)
    g1 = jnp.where(i2 < nrm, gB, 0.0)
    d5 = jnp.where(i1 == nrm, gA, 0.0) + jnp.where(i2 == nrm, gB, 0.0)
    d6 = (jnp.where(i1 == nrm + 1, gA, 0.0)
          + jnp.where(i2 == nrm + 1, gB, 0.0))

    gmeta_ref[...] = (jnp.where(lane == 0, g0, 0.0)
                      + jnp.where(lane == 1, g1, 0.0)
                      + jnp.where(lane == 2, i1.astype(jnp.float32), 0.0)
                      + jnp.where(lane == 3, i2.astype(jnp.float32), 0.0))

    c0 = lg[:, NUM_EXPERTS:NUM_EXPERTS + 1] * 2.0
    c1 = lg[:, NUM_EXPERTS + 1:NUM_EXPERTS + 2] * 2.0
    cm = jnp.maximum(c0, c1)
    e0 = jnp.exp(c0 - cm)
    e1 = jnp.exp(c1 - cm)
    cw0 = e0 / (e0 + e1)
    cw1 = e1 / (e0 + e1)
    x2 = xb * 2.0
    ceo = cw0 * x2 + cw1 * cvec_ref[0:1, :]
    base_ref[...] = d5 * ceo + d6 * x2


def _router_call(xt, wgcat, cvec):
    T, D = xt.shape
    return pl.pallas_call(
        _router_body,
        grid=(T // TOK_BLK,),
        in_specs=[
            pl.BlockSpec((TOK_BLK, D), lambda t: (t, 0)),
            pl.BlockSpec((LANES, D), lambda t: (0, 0)),
            pl.BlockSpec((8, D), lambda t: (0, 0)),
        ],
        out_specs=[
            pl.BlockSpec((TOK_BLK, LANES), lambda t: (t, 0)),
            pl.BlockSpec((TOK_BLK, LANES), lambda t: (t, 0)),
            pl.BlockSpec((TOK_BLK, D), lambda t: (t, 0)),
        ],
        out_shape=[
            jax.ShapeDtypeStruct((T, LANES), jnp.float32),
            jax.ShapeDtypeStruct((T, LANES), jnp.float32),
            jax.ShapeDtypeStruct((T, D), jnp.float32),
        ],
    )(xt, wgcat, cvec)


def _iota16():
    return jax.lax.broadcasted_iota(jnp.int32, (16,), 0)


def _hist_call(e0, e1):
    T = e0.shape[0]
    tpw = T // NW
    mesh = plsc.VectorSubcoreMesh(core_axis_name="c", subcore_axis_name="s")

    @functools.partial(
        pl.kernel, mesh=mesh,
        compiler_params=pltpu.CompilerParams(needs_layout_passes=False),
        out_type=jax.ShapeDtypeStruct((NW, 16), jnp.int32),
        scratch_types=[
            pltpu.VMEM((tpw,), jnp.int32),
            pltpu.VMEM((tpw,), jnp.int32),
            pltpu.VMEM((16,), jnp.int32),
        ],
    )
    def k(e0_hbm, e1_hbm, hist_hbm, e0_v, e1_v, h_v):
        wid = lax.axis_index("s") * NC + lax.axis_index("c")
        base = wid * tpw
        pltpu.sync_copy(e0_hbm.at[pl.ds(base, tpw)], e0_v)
        pltpu.sync_copy(e1_hbm.at[pl.ds(base, tpw)], e1_v)
        one = jnp.ones((16,), jnp.int32)
        zero = jnp.zeros((16,), jnp.int32)
        counts = [jnp.int32(0)] * N_NORMAL
        for j in range(tpw // 16):
            v0 = e0_v[pl.ds(j * 16, 16)]
            v1 = e1_v[pl.ds(j * 16, 16)]
            for e in range(N_NORMAL):
                counts[e] = (counts[e]
                             + jnp.sum(jnp.where(v0 == e, one, zero))
                             + jnp.sum(jnp.where(v1 == e, one, zero)))
        lanes = _iota16()
        hv = jnp.zeros((16,), jnp.int32)
        for e in range(N_NORMAL):
            hv = jnp.where(lanes == e, counts[e], hv)
        h_v[...] = hv
        pltpu.sync_copy(h_v, hist_hbm.at[wid])

    return k(e0, e1)


def _place_call(e0, e1, hist, x3, n_blk_max, m_pad8):
    T = e0.shape[0]
    tpw = T // NW
    g_steps = n_blk_max * (8192 // FF_CHUNK)
    mesh = plsc.VectorSubcoreMesh(core_axis_name="c", subcore_axis_name="s")
    nf = 8192 // FF_CHUNK
    m_dummy = m_pad8 - 8

    @functools.partial(
        pl.kernel, mesh=mesh,
        compiler_params=pltpu.CompilerParams(needs_layout_passes=False),
        out_type=[
            jax.ShapeDtypeStruct((T,), jnp.int32),
            jax.ShapeDtypeStruct((T,), jnp.int32),
            jax.ShapeDtypeStruct((g_steps,), jnp.int32),
            jax.ShapeDtypeStruct((g_steps,), jnp.int32),
            jax.ShapeDtypeStruct((g_steps,), jnp.int32),
            jax.ShapeDtypeStruct((g_steps,), jnp.int32),
            jax.ShapeDtypeStruct((m_pad8,) + x3.shape[1:], jnp.int32),
        ],
        scratch_types=[
            pltpu.VMEM((NW, 16), jnp.int32),
            pltpu.VMEM((tpw,), jnp.int32),
            pltpu.VMEM((tpw,), jnp.int32),
            pltpu.VMEM((tpw // 32, 32), jnp.int32),
            pltpu.VMEM((tpw // 32, 32), jnp.int32),
            pltpu.VMEM((32,) + x3.shape[1:], jnp.int32),
            pltpu.VMEM((g_steps,), jnp.int32),
            pltpu.VMEM((g_steps,), jnp.int32),
            pltpu.VMEM((g_steps,), jnp.int32),
            pltpu.VMEM((g_steps,), jnp.int32),
            pltpu.SemaphoreType.DMA,
        ],
    )
    def k(e0_hbm, e1_hbm, hist_hbm, x_hbm, pos0_hbm, pos1_hbm, se_hbm,
          srb_hbm, sf_hbm, sk_hbm, xs_hbm, hist_v, e0_v, e1_v, p0d, p1d,
          xbuf, sev, srbv, sfv, skv, sem):
        wid = lax.axis_index("s") * NC + lax.axis_index("c")
        tbase = wid * tpw
        pltpu.sync_copy(hist_hbm, hist_v)
        pltpu.sync_copy(e0_hbm.at[pl.ds(tbase, tpw)], e0_v)
        pltpu.sync_copy(e1_hbm.at[pl.ds(tbase, tpw)], e1_v)

        lanes = _iota16()
        widx_a = lanes
        widx_b = lanes + 16
        counts = []
        woff = []
        for e in range(N_NORMAL):
            col = jnp.full((16,), e, jnp.int32)
            ga = plsc.load_gather(hist_v, [widx_a, col])
            gb = plsc.load_gather(hist_v, [widx_b, col])
            counts.append(jnp.sum(ga) + jnp.sum(gb))
            ma = widx_a < wid
            mb = widx_b < wid
            woff.append(jnp.sum(jnp.where(ma, ga, 0))
                        + jnp.sum(jnp.where(mb, gb, 0)))

        nblk = [(counts[e] + (BM - 1)) // BM for e in range(N_NORMAL)]
        cumnb = [jnp.int32(0)]
        for e in range(N_NORMAL):
            cumnb.append(cumnb[e] + nblk[e])
        off = [cumnb[e] * BM + woff[e] for e in range(N_NORMAL)]

        for j in range(tpw // 16):
            for kk in range(2):
                ev = (e0_v if kk == 0 else e1_v)[pl.ds(j * 16, 16)]
                posv = jnp.full((16,), m_dummy, jnp.int32)
                one = jnp.ones((16,), jnp.int32)
                zero = jnp.zeros((16,), jnp.int32)
                for e in range(N_NORMAL):
                    msk = ev == e
                    mi = jnp.where(msk, one, zero)
                    pf = plsc.cumsum(mi)
                    posv = jnp.where(msk, off[e] + pf - 1, posv)
                    off[e] = off[e] + jnp.sum(mi)
                dst = p0d if kk == 0 else p1d
                dst[j // 2, pl.ds((j % 2) * 16, 16)] = posv

        for c in range(tpw // 32):
            pltpu.sync_copy(p0d.at[c], pos0_hbm.at[pl.ds(tbase + c * 32, 32)])
            pltpu.sync_copy(p1d.at[c], pos1_hbm.at[pl.ds(tbase + c * 32, 32)])

        for c in range(tpw // 32):
            pltpu.sync_copy(x_hbm.at[pl.ds(tbase + c * 32, 32)], xbuf)
            cp0 = pltpu.async_copy(xbuf, xs_hbm.at[p0d.at[c]], sem)
            cp1 = pltpu.async_copy(xbuf, xs_hbm.at[p1d.at[c]], sem)
            cp0.wait()
            cp1.wait()

        @pl.when(wid == 0)
        def _steps():
            total = cumnb[N_NORMAL]
            e_last = jnp.int32(0)
            for e in range(1, N_NORMAL):
                e_last = e_last + jnp.where(total - 1 >= cumnb[e], 1, 0)
            rb_last = jnp.maximum(total - 1, 0)
            onev = jnp.ones((16,), jnp.int32)
            zerov = jnp.zeros((16,), jnp.int32)
            for sv in range(g_steps // 16):
                svec = lanes + sv * 16
                ivec = lax.shift_right_logical(svec, 4)
                fvec = lax.bitwise_and(svec, jnp.int32(nf - 1))
                active = ivec < total
                e_i = jnp.zeros((16,), jnp.int32)
                for e in range(1, N_NORMAL):
                    e_i = e_i + jnp.where(ivec >= cumnb[e], onev, zerov)
                sl = pl.ds(sv * 16, 16)
                sev[sl] = jnp.where(active, e_i, e_last)
                srbv[sl] = jnp.where(active, ivec, rb_last)
                sfv[sl] = jnp.where(active, fvec, nf - 1)
                skv[sl] = jnp.where(active, zerov, onev)
            pltpu.sync_copy(sev, se_hbm)
            pltpu.sync_copy(srbv, srb_hbm)
            pltpu.sync_copy(sfv, sf_hbm)
            pltpu.sync_copy(skv, sk_hbm)

    return k(e0, e1, hist, x3)


def _ffn_body(se_ref, srb_ref, sf_ref, sk_ref, x_ref, w1_ref, w2_ref,
              out_ref, acc_ref):
    s = pl.program_id(0)
    f = sf_ref[s]
    nf = 8192 // FF_CHUNK

    @pl.when(sk_ref[s] == 0)
    def _work():
        xb = x_ref[...] * jnp.bfloat16(2.0)
        h = jax.lax.dot_general(xb, w1_ref[0], (((1,), (1,)), ((), ())),
                                preferred_element_type=jnp.float32)
        h = jax.nn.gelu(h)
        hb = h.astype(jnp.bfloat16)
        o = jax.lax.dot_general(hb, w2_ref[0], (((1,), (1,)), ((), ())),
                                preferred_element_type=jnp.float32)

        @pl.when(f == 0)
        def _init():
            acc_ref[...] = o

        @pl.when(f > 0)
        def _acc():
            acc_ref[...] += o

        @pl.when(f == nf - 1)
        def _flush():
            out_ref[...] = acc_ref[...]


def _ffn_call(se, srb, sf, sk, xs2, w1_bf, w2_bf, m_pad8):
    E, F, D = w1_bf.shape
    g_steps = se.shape[0]
    grid_spec = pltpu.PrefetchScalarGridSpec(
        num_scalar_prefetch=4,
        grid=(g_steps,),
        in_specs=[
            pl.BlockSpec((BM, D), lambda s, se, srb, sf, sk: (srb[s], 0)),
            pl.BlockSpec((1, FF_CHUNK, D),
                         lambda s, se, srb, sf, sk: (se[s], sf[s], 0)),
            pl.BlockSpec((1, D, FF_CHUNK),
                         lambda s, se, srb, sf, sk: (se[s], 0, sf[s])),
        ],
        out_specs=pl.BlockSpec((BM, D), lambda s, se, srb, sf, sk: (srb[s], 0)),
        scratch_shapes=[pltpu.VMEM((BM, D), jnp.float32)],
    )
    return pl.pallas_call(
        _ffn_body,
        grid_spec=grid_spec,
        out_shape=jax.ShapeDtypeStruct((m_pad8, D), jnp.float32),
    )(se, srb, sf, sk, xs2, w1_bf, w2_bf)


def _combine_call(pos0, pos1, g0, g1, base, os2):
    T, D = base.shape
    tpw = T // NW
    ch = 16
    nch = tpw // ch
    nvec = D // 16
    mesh = plsc.VectorSubcoreMesh(core_axis_name="c", subcore_axis_name="s")

    @functools.partial(
        pl.kernel, mesh=mesh,
        compiler_params=pltpu.CompilerParams(needs_layout_passes=False),
        out_type=jax.ShapeDtypeStruct((T, D), jnp.float32),
        scratch_types=[
            pltpu.VMEM((tpw,), jnp.int32),
            pltpu.VMEM((tpw,), jnp.int32),
            pltpu.VMEM((tpw,), jnp.float32),
            pltpu.VMEM((tpw,), jnp.float32),
            pltpu.VMEM((ch, D), jnp.float32),
            pltpu.VMEM((ch, D), jnp.float32),
            pltpu.VMEM((ch, D), jnp.float32),
            pltpu.SemaphoreType.DMA,
        ],
    )
    def k(pos0_hbm, pos1_hbm, g0_hbm, g1_hbm, base_hbm, os_hbm, out_hbm,
          p0v, p1v, g0v, g1v, r0, r1, bb, sem):
        wid = lax.axis_index("s") * NC + lax.axis_index("c")
        tbase = wid * tpw
        pltpu.sync_copy(pos0_hbm.at[pl.ds(tbase, tpw)], p0v)
        pltpu.sync_copy(pos1_hbm.at[pl.ds(tbase, tpw)], p1v)
        pltpu.sync_copy(g0_hbm.at[pl.ds(tbase, tpw)], g0v)
        pltpu.sync_copy(g1_hbm.at[pl.ds(tbase, tpw)], g1v)

        def chunk(c, carry):
            start = tbase + c * ch
            idx0 = p0v[pl.ds(c * ch, ch)]
            idx1 = p1v[pl.ds(c * ch, ch)]
            cp0 = pltpu.async_copy(os_hbm.at[idx0], r0, sem)
            cp1 = pltpu.async_copy(os_hbm.at[idx1], r1, sem)
            pltpu.sync_copy(base_hbm.at[pl.ds(start, ch)], bb)
            cp0.wait()
            cp1.wait()
            gvec0 = g0v[pl.ds(c * ch, ch)]
            gvec1 = g1v[pl.ds(c * ch, ch)]
            gs0 = [gvec0[i] for i in range(ch)]
            gs1 = [gvec1[i] for i in range(ch)]
            gb0 = [jnp.full((16,), g, jnp.float32) for g in gs0]
            gb1 = [jnp.full((16,), g, jnp.float32) for g in gs1]
            zero = jnp.zeros((16,), jnp.float32)

            def body(r, carry2):
                sl = pl.ds(r * 16, 16)
                for i in range(ch):
                    v = (bb[i, sl]
                         + jnp.where(gb0[i] == zero, zero, gb0[i] * r0[i, sl])
                         + jnp.where(gb1[i] == zero, zero, gb1[i] * r1[i, sl]))
                    bb[i, sl] = v
                return carry2

            lax.fori_loop(0, nvec, body, 0)
            pltpu.sync_copy(bb, out_hbm.at[pl.ds(start, ch)])
            return carry

        lax.fori_loop(0, nch, chunk, 0)

    return k(pos0, pos1, g0, g1, base, os2)


def kernel(x, wg, w1, w2, const_vec, const_wg):
    B, S, D = x.shape
    T = B * S
    A = 2 * T
    F = w1.shape[1]
    nf = F // FF_CHUNK
    n_blk_max = F // BM // 16 + 21
    n_blk_max = A // BM + (N_NORMAL - 1)
    m_pad = A + N_NORMAL * BM
    m_pad8 = m_pad + 8

    xt = x.reshape(T, D)
    wgcat = jnp.zeros((LANES, D), jnp.float32)
    wgcat = wgcat.at[0:NUM_EXPERTS].set(wg)
    wgcat = wgcat.at[NUM_EXPERTS:NUM_EXPERTS + 2].set(const_wg)
    cvec = jnp.broadcast_to(const_vec[None, :], (8, D))

    logits_pad, gmeta, base = _router_call(xt, wgcat, cvec)
    logits = logits_pad[:, :NUM_EXPERTS]
    g0 = gmeta[:, 0]
    g1 = gmeta[:, 1]
    e0 = gmeta[:, 2].astype(jnp.int32)
    e1 = gmeta[:, 3].astype(jnp.int32)

    x_bf = xt.astype(jnp.bfloat16)
    x_i3 = jax.lax.bitcast_convert_type(
        x_bf.reshape(T, D // 2, 2), jnp.int32).reshape(T, D // 2 // LANES, LANES)
    hist = _hist_call(e0, e1)
    pos0, pos1, se, srb, sf, sk, xs3 = _place_call(
        e0, e1, hist, x_i3, n_blk_max, m_pad8)
    xs2 = jax.lax.bitcast_convert_type(
        xs3.reshape(m_pad8, D // 2), jnp.bfloat16).reshape(m_pad8, D)

    w1_bf = w1.astype(jnp.bfloat16)
    w2_bf = w2.astype(jnp.bfloat16)
    os2 = _ffn_call(se, srb, sf, sk, xs2, w1_bf, w2_bf, m_pad8)

    out = _combine_call(pos0, pos1, g0, g1, base, os2)
    return out.reshape(x.shape), logits


# speedup vs baseline: 3.7789x; 1.6926x over previous
import functools

import jax
import jax.numpy as jnp
from jax import lax
from jax.experimental import pallas as pl
from jax.experimental.pallas import tpu as pltpu
from jax.experimental.pallas import tpu_sc as plsc

NUM_EXPERTS = 8
N_NORMAL = 5
LANES = 128
TOK_BLK = 256
BM = 512
FF_CHUNK = 512
NC, NS = 2, 16
NW = NC * NS


def _router_body(x_ref, wgcat_ref, cvec_ref, logits_ref, gmeta_ref, base_ref):
    xb = x_ref[...]
    lg = jax.lax.dot_general(
        xb, wgcat_ref[...], (((1,), (1,)), ((), ())),
        preferred_element_type=jnp.float32,
        precision=jax.lax.Precision.DEFAULT)
    logits_ref[...] = lg

    lane = jax.lax.broadcasted_iota(jnp.int32, lg.shape, 1)
    valid = lane < NUM_EXPERTS
    l8 = jnp.where(valid, lg, jnp.float32(-1e30))
    m = jnp.max(l8, axis=1, keepdims=True)
    ex = jnp.where(valid, jnp.exp(l8 - m), 0.0)
    p = ex / jnp.sum(ex, axis=1, keepdims=True)

    big = jnp.int32(LANES)
    v1 = jnp.max(p, axis=1, keepdims=True)
    i1 = jnp.min(jnp.where(valid & (p == v1), lane, big), axis=1, keepdims=True)
    cand = jnp.where(valid & (lane != i1), p, -1.0)
    v2 = jnp.max(cand, axis=1, keepdims=True)
    i2 = jnp.min(jnp.where(cand == v2, lane, big), axis=1, keepdims=True)

    zero_id = jnp.int32(NUM_EXPERTS - 1)
    gA = jnp.where(i1 == zero_id, 0.0, v1)
    gB = jnp.where(i2 == zero_id, 0.0, v2)
    ssum = gA + gB
    gA = gA / ssum
    gB = gB / ssum

    nrm = jnp.int32(N_NORMAL)
    g0 = jnp.where(i1 < nrm, gA, 0.0)
    g1 = jnp.where(i2 < nrm, gB, 0.0)
    d5 = jnp.where(i1 == nrm, gA, 0.0) + jnp.where(i2 == nrm, gB, 0.0)
    d6 = (jnp.where(i1 == nrm + 1, gA, 0.0)
          + jnp.where(i2 == nrm + 1, gB, 0.0))

    gmeta_ref[...] = (jnp.where(lane == 0, g0, 0.0)
                      + jnp.where(lane == 1, g1, 0.0)
                      + jnp.where(lane == 2, i1.astype(jnp.float32), 0.0)
                      + jnp.where(lane == 3, i2.astype(jnp.float32), 0.0))

    c0 = lg[:, NUM_EXPERTS:NUM_EXPERTS + 1] * 2.0
    c1 = lg[:, NUM_EXPERTS + 1:NUM_EXPERTS + 2] * 2.0
    cm = jnp.maximum(c0, c1)
    e0 = jnp.exp(c0 - cm)
    e1 = jnp.exp(c1 - cm)
    cw0 = e0 / (e0 + e1)
    cw1 = e1 / (e0 + e1)
    x2 = xb * 2.0
    ceo = cw0 * x2 + cw1 * cvec_ref[0:1, :]
    base_ref[...] = d5 * ceo + d6 * x2


def _router_call(xt, wgcat, cvec):
    T, D = xt.shape
    return pl.pallas_call(
        _router_body,
        grid=(T // TOK_BLK,),
        in_specs=[
            pl.BlockSpec((TOK_BLK, D), lambda t: (t, 0)),
            pl.BlockSpec((LANES, D), lambda t: (0, 0)),
            pl.BlockSpec((8, D), lambda t: (0, 0)),
        ],
        out_specs=[
            pl.BlockSpec((TOK_BLK, LANES), lambda t: (t, 0)),
            pl.BlockSpec((TOK_BLK, LANES), lambda t: (t, 0)),
            pl.BlockSpec((TOK_BLK, D), lambda t: (t, 0)),
        ],
        out_shape=[
            jax.ShapeDtypeStruct((T, LANES), jnp.float32),
            jax.ShapeDtypeStruct((T, LANES), jnp.float32),
            jax.ShapeDtypeStruct((T, D), jnp.float32),
        ],
    )(xt, wgcat, cvec)


def _iota16():
    return jax.lax.broadcasted_iota(jnp.int32, (16,), 0)


def _hist_call(e0, e1):
    T = e0.shape[0]
    tpw = T // NW
    mesh = plsc.VectorSubcoreMesh(core_axis_name="c", subcore_axis_name="s")

    @functools.partial(
        pl.kernel, mesh=mesh,
        compiler_params=pltpu.CompilerParams(needs_layout_passes=False),
        out_type=jax.ShapeDtypeStruct((NW, 16), jnp.int32),
        scratch_types=[
            pltpu.VMEM((tpw,), jnp.int32),
            pltpu.VMEM((tpw,), jnp.int32),
            pltpu.VMEM((16,), jnp.int32),
        ],
    )
    def k(e0_hbm, e1_hbm, hist_hbm, e0_v, e1_v, h_v):
        wid = lax.axis_index("s") * NC + lax.axis_index("c")
        base = wid * tpw
        pltpu.sync_copy(e0_hbm.at[pl.ds(base, tpw)], e0_v)
        pltpu.sync_copy(e1_hbm.at[pl.ds(base, tpw)], e1_v)
        one = jnp.ones((16,), jnp.int32)
        zero = jnp.zeros((16,), jnp.int32)
        counts = [jnp.int32(0)] * N_NORMAL
        for j in range(tpw // 16):
            v0 = e0_v[pl.ds(j * 16, 16)]
            v1 = e1_v[pl.ds(j * 16, 16)]
            for e in range(N_NORMAL):
                counts[e] = (counts[e]
                             + jnp.sum(jnp.where(v0 == e, one, zero))
                             + jnp.sum(jnp.where(v1 == e, one, zero)))
        lanes = _iota16()
        hv = jnp.zeros((16,), jnp.int32)
        for e in range(N_NORMAL):
            hv = jnp.where(lanes == e, counts[e], hv)
        h_v[...] = hv
        pltpu.sync_copy(h_v, hist_hbm.at[wid])

    return k(e0, e1)


def _place_call(e0, e1, hist, x3, n_blk_max, m_pad8):
    T = e0.shape[0]
    tpw = T // NW
    g_steps = n_blk_max * (8192 // FF_CHUNK)
    mesh = plsc.VectorSubcoreMesh(core_axis_name="c", subcore_axis_name="s")
    nf = 8192 // FF_CHUNK
    m_dummy = m_pad8 - 8

    @functools.partial(
        pl.kernel, mesh=mesh,
        compiler_params=pltpu.CompilerParams(needs_layout_passes=False),
        out_type=[
            jax.ShapeDtypeStruct((T,), jnp.int32),
            jax.ShapeDtypeStruct((T,), jnp.int32),
            jax.ShapeDtypeStruct((g_steps,), jnp.int32),
            jax.ShapeDtypeStruct((g_steps,), jnp.int32),
            jax.ShapeDtypeStruct((g_steps,), jnp.int32),
            jax.ShapeDtypeStruct((g_steps,), jnp.int32),
            jax.ShapeDtypeStruct((m_pad8,) + x3.shape[1:], jnp.float32),
        ],
        scratch_types=[
            pltpu.VMEM((NW, 16), jnp.int32),
            pltpu.VMEM((tpw,), jnp.int32),
            pltpu.VMEM((tpw,), jnp.int32),
            pltpu.VMEM((tpw // 32, 32), jnp.int32),
            pltpu.VMEM((tpw // 32, 32), jnp.int32),
            pltpu.VMEM((32,) + x3.shape[1:], jnp.float32),
            pltpu.VMEM((g_steps,), jnp.int32),
            pltpu.VMEM((g_steps,), jnp.int32),
            pltpu.VMEM((g_steps,), jnp.int32),
            pltpu.VMEM((g_steps,), jnp.int32),
            pltpu.SemaphoreType.DMA,
        ],
    )
    def k(e0_hbm, e1_hbm, hist_hbm, x_hbm, pos0_hbm, pos1_hbm, se_hbm,
          srb_hbm, sf_hbm, sk_hbm, xs_hbm, hist_v, e0_v, e1_v, p0d, p1d,
          xbuf, sev, srbv, sfv, skv, sem):
        wid = lax.axis_index("s") * NC + lax.axis_index("c")
        tbase = wid * tpw
        pltpu.sync_copy(hist_hbm, hist_v)
        pltpu.sync_copy(e0_hbm.at[pl.ds(tbase, tpw)], e0_v)
        pltpu.sync_copy(e1_hbm.at[pl.ds(tbase, tpw)], e1_v)

        lanes = _iota16()
        widx_a = lanes
        widx_b = lanes + 16
        counts = []
        woff = []
        for e in range(N_NORMAL):
            col = jnp.full((16,), e, jnp.int32)
            ga = plsc.load_gather(hist_v, [widx_a, col])
            gb = plsc.load_gather(hist_v, [widx_b, col])
            counts.append(jnp.sum(ga) + jnp.sum(gb))
            ma = widx_a < wid
            mb = widx_b < wid
            woff.append(jnp.sum(jnp.where(ma, ga, 0))
                        + jnp.sum(jnp.where(mb, gb, 0)))

        nblk = [(counts[e] + (BM - 1)) // BM for e in range(N_NORMAL)]
        cumnb = [jnp.int32(0)]
        for e in range(N_NORMAL):
            cumnb.append(cumnb[e] + nblk[e])
        off = [cumnb[e] * BM + woff[e] for e in range(N_NORMAL)]

        for j in range(tpw // 16):
            for kk in range(2):
                ev = (e0_v if kk == 0 else e1_v)[pl.ds(j * 16, 16)]
                posv = jnp.full((16,), m_dummy, jnp.int32)
                one = jnp.ones((16,), jnp.int32)
                zero = jnp.zeros((16,), jnp.int32)
                for e in range(N_NORMAL):
                    msk = ev == e
                    mi = jnp.where(msk, one, zero)
                    pf = plsc.cumsum(mi)
                    posv = jnp.where(msk, off[e] + pf - 1, posv)
                    off[e] = off[e] + jnp.sum(mi)
                dst = p0d if kk == 0 else p1d
                dst[j // 2, pl.ds((j % 2) * 16, 16)] = posv

        for c in range(tpw // 32):
            pltpu.sync_copy(p0d.at[c], pos0_hbm.at[pl.ds(tbase + c * 32, 32)])
            pltpu.sync_copy(p1d.at[c], pos1_hbm.at[pl.ds(tbase + c * 32, 32)])

        for c in range(tpw // 32):
            pltpu.sync_copy(x_hbm.at[pl.ds(tbase + c * 32, 32)], xbuf)
            cp0 = pltpu.async_copy(xbuf, xs_hbm.at[p0d.at[c]], sem)
            cp1 = pltpu.async_copy(xbuf, xs_hbm.at[p1d.at[c]], sem)
            cp0.wait()
            cp1.wait()

        @pl.when(wid == 0)
        def _steps():
            total = cumnb[N_NORMAL]
            e_last = jnp.int32(0)
            for e in range(1, N_NORMAL):
                e_last = e_last + jnp.where(total - 1 >= cumnb[e], 1, 0)
            rb_last = jnp.maximum(total - 1, 0)
            onev = jnp.ones((16,), jnp.int32)
            zerov = jnp.zeros((16,), jnp.int32)
            for sv in range(g_steps // 16):
                svec = lanes + sv * 16
                ivec = lax.shift_right_logical(svec, 4)
                fvec = lax.bitwise_and(svec, jnp.int32(nf - 1))
                active = ivec < total
                e_i = jnp.zeros((16,), jnp.int32)
                for e in range(1, N_NORMAL):
                    e_i = e_i + jnp.where(ivec >= cumnb[e], onev, zerov)
                sl = pl.ds(sv * 16, 16)
                sev[sl] = jnp.where(active, e_i, e_last)
                srbv[sl] = jnp.where(active, ivec, rb_last)
                sfv[sl] = jnp.where(active, fvec, nf - 1)
                skv[sl] = jnp.where(active, zerov, onev)
            pltpu.sync_copy(sev, se_hbm)
            pltpu.sync_copy(srbv, srb_hbm)
            pltpu.sync_copy(sfv, sf_hbm)
            pltpu.sync_copy(skv, sk_hbm)

    return k(e0, e1, hist, x3)


def _ffn_body(se_ref, srb_ref, sf_ref, sk_ref, x_ref, w1_ref, w2_ref,
              out_ref, acc_ref):
    s = pl.program_id(0)
    f = sf_ref[s]
    nf = 8192 // FF_CHUNK

    @pl.when(sk_ref[s] == 0)
    def _work():
        xb = (x_ref[...] * 2.0).astype(jnp.bfloat16)
        w1c = w1_ref[0].astype(jnp.bfloat16)
        h = jax.lax.dot_general(xb, w1c, (((1,), (1,)), ((), ())),
                                preferred_element_type=jnp.float32)
        h = jax.nn.gelu(h)
        hb = h.astype(jnp.bfloat16)
        w2c = w2_ref[0].astype(jnp.bfloat16)
        o = jax.lax.dot_general(hb, w2c, (((1,), (1,)), ((), ())),
                                preferred_element_type=jnp.float32)

        @pl.when(f == 0)
        def _init():
            acc_ref[...] = o

        @pl.when(f > 0)
        def _acc():
            acc_ref[...] += o

        @pl.when(f == nf - 1)
        def _flush():
            out_ref[...] = acc_ref[...]


def _ffn_call(se, srb, sf, sk, xs2, w1_bf, w2_bf, m_pad8):
    E, F, D = w1_bf.shape
    g_steps = se.shape[0]
    grid_spec = pltpu.PrefetchScalarGridSpec(
        num_scalar_prefetch=4,
        grid=(g_steps,),
        in_specs=[
            pl.BlockSpec((BM, D), lambda s, se, srb, sf, sk: (srb[s], 0)),
            pl.BlockSpec((1, FF_CHUNK, D),
                         lambda s, se, srb, sf, sk: (se[s], sf[s], 0)),
            pl.BlockSpec((1, D, FF_CHUNK),
                         lambda s, se, srb, sf, sk: (se[s], 0, sf[s])),
        ],
        out_specs=pl.BlockSpec((BM, D), lambda s, se, srb, sf, sk: (srb[s], 0)),
        scratch_shapes=[pltpu.VMEM((BM, D), jnp.float32)],
    )
    return pl.pallas_call(
        _ffn_body,
        grid_spec=grid_spec,
        out_shape=jax.ShapeDtypeStruct((m_pad8, D), jnp.float32),
    )(se, srb, sf, sk, xs2, w1_bf, w2_bf)


def _combine_call(pos0, pos1, g0, g1, base, os2):
    T, D = base.shape
    tpw = T // NW
    ch = 16
    nch = tpw // ch
    nvec = D // 16
    mesh = plsc.VectorSubcoreMesh(core_axis_name="c", subcore_axis_name="s")

    @functools.partial(
        pl.kernel, mesh=mesh,
        compiler_params=pltpu.CompilerParams(needs_layout_passes=False),
        out_type=jax.ShapeDtypeStruct((T, D), jnp.float32),
        scratch_types=[
            pltpu.VMEM((tpw,), jnp.int32),
            pltpu.VMEM((tpw,), jnp.int32),
            pltpu.VMEM((tpw,), jnp.float32),
            pltpu.VMEM((tpw,), jnp.float32),
            pltpu.VMEM((ch, D), jnp.float32),
            pltpu.VMEM((ch, D), jnp.float32),
            pltpu.VMEM((ch, D), jnp.float32),
            pltpu.SemaphoreType.DMA,
        ],
    )
    def k(pos0_hbm, pos1_hbm, g0_hbm, g1_hbm, base_hbm, os_hbm, out_hbm,
          p0v, p1v, g0v, g1v, r0, r1, bb, sem):
        wid = lax.axis_index("s") * NC + lax.axis_index("c")
        tbase = wid * tpw
        pltpu.sync_copy(pos0_hbm.at[pl.ds(tbase, tpw)], p0v)
        pltpu.sync_copy(pos1_hbm.at[pl.ds(tbase, tpw)], p1v)
        pltpu.sync_copy(g0_hbm.at[pl.ds(tbase, tpw)], g0v)
        pltpu.sync_copy(g1_hbm.at[pl.ds(tbase, tpw)], g1v)

        def chunk(c, carry):
            start = tbase + c * ch
            idx0 = p0v[pl.ds(c * ch, ch)]
            idx1 = p1v[pl.ds(c * ch, ch)]
            cp0 = pltpu.async_copy(os_hbm.at[idx0], r0, sem)
            cp1 = pltpu.async_copy(os_hbm.at[idx1], r1, sem)
            pltpu.sync_copy(base_hbm.at[pl.ds(start, ch)], bb)
            cp0.wait()
            cp1.wait()
            gvec0 = g0v[pl.ds(c * ch, ch)]
            gvec1 = g1v[pl.ds(c * ch, ch)]
            gs0 = [gvec0[i] for i in range(ch)]
            gs1 = [gvec1[i] for i in range(ch)]
            gb0 = [jnp.full((16,), g, jnp.float32) for g in gs0]
            gb1 = [jnp.full((16,), g, jnp.float32) for g in gs1]
            zero = jnp.zeros((16,), jnp.float32)

            def body(r, carry2):
                sl = pl.ds(r * 16, 16)
                for i in range(ch):
                    v = (bb[i, sl]
                         + jnp.where(gb0[i] == zero, zero, gb0[i] * r0[i, sl])
                         + jnp.where(gb1[i] == zero, zero, gb1[i] * r1[i, sl]))
                    bb[i, sl] = v
                return carry2

            lax.fori_loop(0, nvec, body, 0)
            pltpu.sync_copy(bb, out_hbm.at[pl.ds(start, ch)])
            return carry

        lax.fori_loop(0, nch, chunk, 0)

    return k(pos0, pos1, g0, g1, base, os2)


def kernel(x, wg, w1, w2, const_vec, const_wg):
    B, S, D = x.shape
    T = B * S
    A = 2 * T
    F = w1.shape[1]
    nf = F // FF_CHUNK
    n_blk_max = F // BM // 16 + 21
    n_blk_max = A // BM + (N_NORMAL - 1)
    m_pad = A + N_NORMAL * BM
    m_pad8 = m_pad + 8

    xt = x.reshape(T, D)
    wgcat = jnp.zeros((LANES, D), jnp.float32)
    wgcat = wgcat.at[0:NUM_EXPERTS].set(wg)
    wgcat = wgcat.at[NUM_EXPERTS:NUM_EXPERTS + 2].set(const_wg)
    cvec = jnp.broadcast_to(const_vec[None, :], (8, D))

    logits_pad, gmeta, base = _router_call(xt, wgcat, cvec)
    logits = logits_pad[:, :NUM_EXPERTS]
    g0 = gmeta[:, 0]
    g1 = gmeta[:, 1]
    e0 = gmeta[:, 2].astype(jnp.int32)
    e1 = gmeta[:, 3].astype(jnp.int32)

    x_f3 = xt.reshape(T, D // LANES, LANES)
    hist = _hist_call(e0, e1)
    pos0, pos1, se, srb, sf, sk, xs3 = _place_call(
        e0, e1, hist, x_f3, n_blk_max, m_pad8)
    xs2 = xs3.reshape(m_pad8, D)

    os2 = _ffn_call(se, srb, sf, sk, xs2, w1, w2, m_pad8)

    out = _combine_call(pos0, pos1, g0, g1, base, os2)
    return out.reshape(x.shape), logits
